# Initial kernel scaffold; baseline (speedup 1.0000x reference)
import jax
import jax.numpy as jnp
from jax.experimental import pallas as pl


def kernel(x, pe):
    raise NotImplementedError("write your pallas kernel here")


# speedup vs baseline: 1.9147x; 1.9147x over previous
import math

import jax
import jax.numpy as jnp
from jax.experimental import pallas as pl

_D = 1024
_BS = 512


def _add_pe_kernel(x_ref, pe_ref, o_ref, *, inv_scale):
    o_ref[...] = x_ref[...] + pe_ref[...] * inv_scale


def kernel(x, pe):
    b, l, d = x.shape
    inv_scale = 1.0 / math.sqrt(d)
    pe_l = pe[:l]
    import functools
    grid = (l // _BS, b)
    return pl.pallas_call(
        functools.partial(_add_pe_kernel, inv_scale=inv_scale),
        grid=grid,
        in_specs=[
            pl.BlockSpec((1, _BS, d), lambda i, j: (j, i, 0)),
            pl.BlockSpec((_BS, d), lambda i, j: (i, 0)),
        ],
        out_specs=pl.BlockSpec((1, _BS, d), lambda i, j: (j, i, 0)),
        out_shape=jax.ShapeDtypeStruct((b, l, d), x.dtype),
    )(x, pe_l)


# speedup vs baseline: 2.1051x; 1.0995x over previous
import math

import jax
import jax.numpy as jnp
from jax.experimental import pallas as pl

_D = 1024
_BS = 1024


def _add_pe_kernel(x_ref, pe_ref, o_ref, *, inv_scale):
    o_ref[...] = x_ref[...] + pe_ref[...] * inv_scale


def kernel(x, pe):
    b, l, d = x.shape
    inv_scale = 1.0 / math.sqrt(d)
    pe_l = pe[:l]
    import functools
    grid = (l // _BS, b)
    return pl.pallas_call(
        functools.partial(_add_pe_kernel, inv_scale=inv_scale),
        grid=grid,
        in_specs=[
            pl.BlockSpec((1, _BS, d), lambda i, j: (j, i, 0)),
            pl.BlockSpec((_BS, d), lambda i, j: (i, 0)),
        ],
        out_specs=pl.BlockSpec((1, _BS, d), lambda i, j: (j, i, 0)),
        out_shape=jax.ShapeDtypeStruct((b, l, d), x.dtype),
    )(x, pe_l)


# speedup vs baseline: 2.2727x; 1.0796x over previous
import math

import jax
import jax.numpy as jnp
from jax.experimental import pallas as pl

_D = 1024
_BS = 2048


def _add_pe_kernel(x_ref, pe_ref, o_ref, *, inv_scale):
    o_ref[...] = x_ref[...] + pe_ref[...] * inv_scale


def kernel(x, pe):
    b, l, d = x.shape
    inv_scale = 1.0 / math.sqrt(d)
    pe_l = pe[:l]
    import functools
    grid = (l // _BS, b)
    return pl.pallas_call(
        functools.partial(_add_pe_kernel, inv_scale=inv_scale),
        grid=grid,
        in_specs=[
            pl.BlockSpec((1, _BS, d), lambda i, j: (j, i, 0)),
            pl.BlockSpec((_BS, d), lambda i, j: (i, 0)),
        ],
        out_specs=pl.BlockSpec((1, _BS, d), lambda i, j: (j, i, 0)),
        out_shape=jax.ShapeDtypeStruct((b, l, d), x.dtype),
    )(x, pe_l)


# speedup vs baseline: 2.2862x; 1.0059x over previous
import math

import jax
import jax.numpy as jnp
from jax.experimental import pallas as pl
from jax.experimental.pallas import tpu as pltpu

_D = 1024
_BS = 2048


def _add_pe_kernel(x_ref, pe_ref, o_ref, *, inv_scale):
    o_ref[...] = x_ref[...] + pe_ref[...] * inv_scale


def kernel(x, pe):
    b, l, d = x.shape
    inv_scale = 1.0 / math.sqrt(d)
    pe_l = pe[:l]
    import functools
    grid = (l // _BS, b)
    return pl.pallas_call(
        functools.partial(_add_pe_kernel, inv_scale=inv_scale),
        grid=grid,
        in_specs=[
            pl.BlockSpec((1, _BS, d), lambda i, j: (j, i, 0)),
            pl.BlockSpec((_BS, d), lambda i, j: (i, 0)),
        ],
        out_specs=pl.BlockSpec((1, _BS, d), lambda i, j: (j, i, 0)),
        out_shape=jax.ShapeDtypeStruct((b, l, d), x.dtype),
        compiler_params=pltpu.CompilerParams(
            dimension_semantics=("parallel", "parallel"),
        ),
    )(x, pe_l)
